# baseline (speedup 1.0000x reference)
import functools

import jax
import jax.numpy as jnp
from jax import lax
from jax.experimental import pallas as pl
from jax.experimental.pallas import tpu as pltpu
from jax.experimental.pallas import tpu_sc as plsc

_F = 16
_KH = _KW = 3
_H = _W = 64
_C = 16
_K = _KH * _KW * _C
_N = _H * _W
_NW = 32
_RPW = _N // _NW
_CH = 32
_BIG = 1e6
_L = 16


def _cmpex(a, b):
    ka, va = a
    kb, vb = b
    c = ka <= kb
    return ((jnp.where(c, ka, kb), jnp.where(c, va, vb)),
            (jnp.where(c, kb, ka), jnp.where(c, vb, va)))


def _bitonic_merge(seq):
    n = len(seq)
    d = n // 2
    while d >= 1:
        for start in range(0, n, 2 * d):
            for i in range(start, start + d):
                lo, hi = _cmpex(seq[i], seq[i + d])
                seq[i], seq[i + d] = lo, hi
        d //= 2
    return [plsc.sort_key_val(k, v) for (k, v) in seq]


def _merge(a, b):
    brev = [(lax.rev(k, (0,)), lax.rev(v, (0,))) for (k, v) in reversed(b)]
    return _bitonic_merge(a + brev)


def _sc_body(p_hbm, j_hbm, out_hbm, jv, pv, ov, skey, sidx, sem):
    del sem
    nc = jnp.int32(2)
    wid = lax.axis_index("s") * nc + lax.axis_index("c")
    base = wid * jnp.int32(_RPW)
    iota = lax.broadcasted_iota(jnp.int32, (_L,), 0)

    pltpu.sync_copy(j_hbm, jv)

    def do_chunk(ch, carry):
        rbase = base + ch * jnp.int32(_CH)
        pltpu.sync_copy(p_hbm.at[pl.ds(rbase * jnp.int32(_K), _CH * _K)], pv)

        def do_row(rr, carry2):
            runs = []
            for i in range(9):
                kv = pv[pl.ds(rr * jnp.int32(_K) + jnp.int32(16 * i), 16)]
                ks, vs = plsc.sort_key_val(kv, iota + 16 * i)
                runs.append([(ks, vs)])
            lvl = [_merge(runs[i], runs[i + 1]) for i in range(0, 8, 2)]
            lvl = [_merge(lvl[0], lvl[1]), _merge(lvl[2], lvl[3])]
            m8 = _merge(lvl[0], lvl[1])
            r8k, r8v = runs[8][0]
            lo7, hi7 = _cmpex(m8[7], (lax.rev(r8k, (0,)), lax.rev(r8v, (0,))))
            lower = _bitonic_merge(m8[:7] + [lo7])
            top = plsc.sort_key_val(hi7[0], hi7[1])
            srt = lower + [top]
            for i in range(9):
                skey[pl.ds(16 * i, 16)] = srt[i][0]
                sidx[pl.ds(16 * i, 16)] = srt[i][1]
            skey[pl.ds(144, 16)] = jnp.full((_L,), _BIG, jnp.float32)

            zero = jnp.zeros((_L,), jnp.float32)

            def group(g, c3):
                s, dm, out, fired = c3
                g16 = g * jnp.int32(16)
                kvv = skey[pl.ds(g16, 16)]
                knx = skey[pl.ds(g16 + jnp.int32(1), 16)]
                ovv = sidx[pl.ds(g16, 16)]
                isfirst = g == jnp.int32(0)
                for l in range(16):
                    jr = plsc.load_gather(
                        jv, [ovv[l] * jnp.int32(_F) + iota])
                    s = s + jr * kvv[l]
                    dm = dm + jr
                    d = dm + 1.0
                    ti = s / d
                    cond = (ti < knx[l]) & (d > 0.0)
                    take = cond & (fired == 0.0)
                    if l == 0:
                        take = take | isfirst
                    out = jnp.where(take, ti, out)
                    fired = jnp.where(cond, jnp.float32(1.0), fired)
                return s, dm, out, fired

            _, _, out, _ = lax.fori_loop(
                jnp.int32(0), jnp.int32(9), group,
                (zero, zero, zero, zero))
            out = jnp.where(out <= 1.0, out, 1.0)
            ov[pl.ds(rr * jnp.int32(_F), _F)] = out
            return carry2

        lax.fori_loop(jnp.int32(0), jnp.int32(_CH), do_row, jnp.int32(0))
        pltpu.sync_copy(ov, out_hbm.at[pl.ds(rbase * jnp.int32(_F), _CH * _F)])
        return carry

    lax.fori_loop(jnp.int32(0), jnp.int32(_RPW // _CH), do_chunk, jnp.int32(0))


_mesh = plsc.VectorSubcoreMesh(core_axis_name="c", subcore_axis_name="s")

_sc_kernel = functools.partial(
    pl.kernel,
    out_type=jax.ShapeDtypeStruct((_N * _F,), jnp.float32),
    mesh=_mesh,
    compiler_params=pltpu.CompilerParams(needs_layout_passes=False),
    scratch_types=[
        pltpu.VMEM((_K * _F,), jnp.float32),
        pltpu.VMEM((_CH * _K,), jnp.float32),
        pltpu.VMEM((_CH * _F,), jnp.float32),
        pltpu.VMEM((160,), jnp.float32),
        pltpu.VMEM((160,), jnp.int32),
        pltpu.SemaphoreType.DMA,
    ],
)(_sc_body)


def kernel(tj, kernel):
    x = tj[0].astype(jnp.float32)
    xp = jnp.pad(x, ((1, 1), (1, 1), (0, 0)))
    parts = [xp[i:i + _H, j:j + _W, :] for i in range(_KH) for j in range(_KW)]
    patches = jnp.concatenate(parts, axis=-1).reshape(_N * _K)
    J = kernel.reshape(_K * _F).astype(jnp.float32)
    out = _sc_kernel(patches, J)
    return out.reshape(1, _H, _W, _F).astype(jnp.float64)


# speedup vs baseline: 1.0765x; 1.0765x over previous
import functools

import jax
import jax.numpy as jnp
from jax import lax
from jax.experimental import pallas as pl
from jax.experimental.pallas import tpu as pltpu
from jax.experimental.pallas import tpu_sc as plsc

_F = 16
_KH = _KW = 3
_H = _W = 64
_C = 16
_K = _KH * _KW * _C
_N = _H * _W
_NW = 32
_RPW = _N // _NW
_CH = 32
_BIG = 1e6
_L = 16


def _cmpex(a, b):
    ka, va = a
    kb, vb = b
    c = ka <= kb
    return ((jnp.where(c, ka, kb), jnp.where(c, va, vb)),
            (jnp.where(c, kb, ka), jnp.where(c, vb, va)))


def _bitonic_merge(seq):
    n = len(seq)
    d = n // 2
    while d >= 1:
        for start in range(0, n, 2 * d):
            for i in range(start, start + d):
                lo, hi = _cmpex(seq[i], seq[i + d])
                seq[i], seq[i + d] = lo, hi
        d //= 2
    return [plsc.sort_key_val(k, v) for (k, v) in seq]


def _merge(a, b):
    brev = [(lax.rev(k, (0,)), lax.rev(v, (0,))) for (k, v) in reversed(b)]
    return _bitonic_merge(a + brev)


def _sc_body(p_hbm, j_hbm, out_hbm, jv, pv, ov, skey, sidx, sem):
    del sem
    nc = jnp.int32(2)
    wid = lax.axis_index("s") * nc + lax.axis_index("c")
    base = wid * jnp.int32(_RPW)
    iota = lax.broadcasted_iota(jnp.int32, (_L,), 0)

    pltpu.sync_copy(j_hbm, jv)

    def do_chunk(ch, carry):
        rbase = base + ch * jnp.int32(_CH)
        pltpu.sync_copy(p_hbm.at[pl.ds(rbase * jnp.int32(_K), _CH * _K)], pv)

        def do_row(rr, carry2):
            runs = []
            for i in range(9):
                kv = pv[pl.ds(rr * jnp.int32(_K) + jnp.int32(16 * i), 16)]
                ks, vs = plsc.sort_key_val(kv, iota + 16 * i)
                runs.append([(ks, vs)])
            lvl = [_merge(runs[i], runs[i + 1]) for i in range(0, 8, 2)]
            lvl = [_merge(lvl[0], lvl[1]), _merge(lvl[2], lvl[3])]
            m8 = _merge(lvl[0], lvl[1])
            r8k, r8v = runs[8][0]
            lo7, hi7 = _cmpex(m8[7], (lax.rev(r8k, (0,)), lax.rev(r8v, (0,))))
            lower = _bitonic_merge(m8[:7] + [lo7])
            top = plsc.sort_key_val(hi7[0], hi7[1])
            srt = lower + [top]
            for i in range(9):
                skey[pl.ds(16 * i, 16)] = srt[i][0]
                sidx[pl.ds(16 * i, 16)] = srt[i][1]
            skey[pl.ds(144, 16)] = jnp.full((_L,), _BIG, jnp.float32)

            zero = jnp.zeros((_L,), jnp.float32)
            one = jnp.full((_L,), 1.0, jnp.float32)

            def group(g, c5):
                s, d, outs, outd, fired = c5
                g16 = g * jnp.int32(16)
                kvv = skey[pl.ds(g16, 16)]
                knx = skey[pl.ds(g16 + jnp.int32(1), 16)]
                ovv = sidx[pl.ds(g16, 16)]
                isfirst = g == jnp.int32(0)
                for l in range(16):
                    jr = plsc.load_gather(
                        jv, [ovv[l] * jnp.int32(_F) + iota])
                    s = s + jr * kvv[l]
                    d = d + jr
                    cond = (s < knx[l] * d) & (d > 0.0)
                    take = cond & (fired == 0.0)
                    if l == 0:
                        take = take | isfirst
                    outs = jnp.where(take, s, outs)
                    outd = jnp.where(take, d, outd)
                    fired = jnp.where(cond, jnp.float32(1.0), fired)
                return s, d, outs, outd, fired

            _, _, outs, outd, _ = lax.fori_loop(
                jnp.int32(0), jnp.int32(9), group,
                (zero, one, zero, one, zero))
            out = outs / outd
            out = jnp.where(out <= 1.0, out, 1.0)
            ov[pl.ds(rr * jnp.int32(_F), _F)] = out
            return carry2

        lax.fori_loop(jnp.int32(0), jnp.int32(_CH), do_row, jnp.int32(0))
        pltpu.sync_copy(ov, out_hbm.at[pl.ds(rbase * jnp.int32(_F), _CH * _F)])
        return carry

    lax.fori_loop(jnp.int32(0), jnp.int32(_RPW // _CH), do_chunk, jnp.int32(0))


_mesh = plsc.VectorSubcoreMesh(core_axis_name="c", subcore_axis_name="s")

_sc_kernel = functools.partial(
    pl.kernel,
    out_type=jax.ShapeDtypeStruct((_N * _F,), jnp.float32),
    mesh=_mesh,
    compiler_params=pltpu.CompilerParams(needs_layout_passes=False),
    scratch_types=[
        pltpu.VMEM((_K * _F,), jnp.float32),
        pltpu.VMEM((_CH * _K,), jnp.float32),
        pltpu.VMEM((_CH * _F,), jnp.float32),
        pltpu.VMEM((160,), jnp.float32),
        pltpu.VMEM((160,), jnp.int32),
        pltpu.SemaphoreType.DMA,
    ],
)(_sc_body)


def kernel(tj, kernel):
    x = tj[0].astype(jnp.float32)
    xp = jnp.pad(x, ((1, 1), (1, 1), (0, 0)))
    parts = [xp[i:i + _H, j:j + _W, :] for i in range(_KH) for j in range(_KW)]
    patches = jnp.concatenate(parts, axis=-1).reshape(_N * _K)
    J = kernel.reshape(_K * _F).astype(jnp.float32)
    out = _sc_kernel(patches, J)
    return out.reshape(1, _H, _W, _F).astype(jnp.float64)


# speedup vs baseline: 1.2376x; 1.1497x over previous
import functools

import jax
import jax.numpy as jnp
from jax import lax
from jax.experimental import pallas as pl
from jax.experimental.pallas import tpu as pltpu
from jax.experimental.pallas import tpu_sc as plsc

_F = 16
_KH = _KW = 3
_H = _W = 64
_C = 16
_K = _KH * _KW * _C
_N = _H * _W
_NW = 32
_RPW = _N // _NW
_CH = 32
_NT = _KH * _KW
_BIG = 1e6
_L = 16
_ZROW = _N


def _cmpex(a, b):
    ka, va = a
    kb, vb = b
    c = ka <= kb
    return ((jnp.where(c, ka, kb), jnp.where(c, va, vb)),
            (jnp.where(c, kb, ka), jnp.where(c, vb, va)))


def _bitonic_merge(seq):
    n = len(seq)
    d = n // 2
    while d >= 1:
        for start in range(0, n, 2 * d):
            for i in range(start, start + d):
                lo, hi = _cmpex(seq[i], seq[i + d])
                seq[i], seq[i + d] = lo, hi
        d //= 2
    return [plsc.sort_key_val(k, v) for (k, v) in seq]


def _merge(a, b):
    brev = [(lax.rev(k, (0,)), lax.rev(v, (0,))) for (k, v) in reversed(b)]
    return _bitonic_merge(a + brev)


def _sc_body(tj_hbm, j_hbm, out_hbm, jv, ptap, ov, skey, sidx,
             ix0, ix1, ix2, sem):
    nc = jnp.int32(2)
    wid = lax.axis_index("s") * nc + lax.axis_index("c")
    base = wid * jnp.int32(_RPW)
    iota = lax.broadcasted_iota(jnp.int32, (_L,), 0)

    pltpu.sync_copy(j_hbm, jv)
    ixbufs = (ix0, ix1, ix2)

    def do_chunk(ch, carry):
        rbase = base + ch * jnp.int32(_CH)

        for t in range(_NT):
            dy, dx = t // _KW - 1, t % _KW - 1
            for h in range(2):
                p = rbase + iota + jnp.int32(16 * h)
                py = lax.shift_right_logical(p, jnp.int32(6)) + jnp.int32(dy)
                px = (p & jnp.int32(63)) + jnp.int32(dx)
                oob = ((py < jnp.int32(0)) | (py > jnp.int32(_H - 1))
                       | (px < jnp.int32(0)) | (px > jnp.int32(_W - 1)))
                ridx = p + jnp.int32(dy * _W + dx)
                idx = jnp.where(oob, jnp.int32(_ZROW), ridx)
                pos = t * _CH + h * _L
                ixbufs[pos // 96][pl.ds(pos % 96, 16)] = idx
        cps = [pltpu.make_async_copy(
            tj_hbm.at[ixbufs[b]], ptap.at[pl.ds(96 * b, 96)], sem)
            for b in range(3)]
        for cp in cps:
            cp.start()
        for cp in cps:
            cp.wait()

        def do_row(rr, carry2):
            runs = []
            for i in range(9):
                kv = ptap[rr + jnp.int32(i * _CH), :]
                ks, vs = plsc.sort_key_val(kv, iota + 16 * i)
                runs.append([(ks, vs)])
            lvl = [_merge(runs[i], runs[i + 1]) for i in range(0, 8, 2)]
            lvl = [_merge(lvl[0], lvl[1]), _merge(lvl[2], lvl[3])]
            m8 = _merge(lvl[0], lvl[1])
            r8k, r8v = runs[8][0]
            lo7, hi7 = _cmpex(m8[7], (lax.rev(r8k, (0,)), lax.rev(r8v, (0,))))
            lower = _bitonic_merge(m8[:7] + [lo7])
            top = plsc.sort_key_val(hi7[0], hi7[1])
            srt = lower + [top]
            for i in range(9):
                skey[pl.ds(16 * i, 16)] = srt[i][0]
                sidx[pl.ds(16 * i, 16)] = srt[i][1]
            skey[pl.ds(144, 16)] = jnp.full((_L,), _BIG, jnp.float32)

            zero = jnp.zeros((_L,), jnp.float32)
            one = jnp.full((_L,), 1.0, jnp.float32)

            def group(g, c5):
                s, d, outs, outd, fired = c5
                g16 = g * jnp.int32(16)
                kvv = skey[pl.ds(g16, 16)]
                knx = skey[pl.ds(g16 + jnp.int32(1), 16)]
                ovv = sidx[pl.ds(g16, 16)]
                isfirst = g == jnp.int32(0)
                for l in range(16):
                    jr = plsc.load_gather(
                        jv, [ovv[l] * jnp.int32(_F) + iota])
                    s = s + jr * kvv[l]
                    d = d + jr
                    cond = (s < knx[l] * d) & (d > 0.0)
                    take = cond & (fired == 0.0)
                    if l == 0:
                        take = take | isfirst
                    outs = jnp.where(take, s, outs)
                    outd = jnp.where(take, d, outd)
                    fired = jnp.where(cond, jnp.float32(1.0), fired)
                return s, d, outs, outd, fired

            _, _, outs, outd, _ = lax.fori_loop(
                jnp.int32(0), jnp.int32(9), group,
                (zero, one, zero, one, zero))
            out = outs / outd
            out = jnp.where(out <= 1.0, out, 1.0)
            ov[pl.ds(rr * jnp.int32(_F), _F)] = out
            return carry2

        lax.fori_loop(jnp.int32(0), jnp.int32(_CH), do_row, jnp.int32(0))
        pltpu.sync_copy(ov, out_hbm.at[pl.ds(rbase * jnp.int32(_F), _CH * _F)])
        return carry

    lax.fori_loop(jnp.int32(0), jnp.int32(_RPW // _CH), do_chunk, jnp.int32(0))


_mesh = plsc.VectorSubcoreMesh(core_axis_name="c", subcore_axis_name="s")

_sc_kernel = functools.partial(
    pl.kernel,
    out_type=jax.ShapeDtypeStruct((_N * _F,), jnp.float32),
    mesh=_mesh,
    compiler_params=pltpu.CompilerParams(needs_layout_passes=False, use_tc_tiling_on_sc=False),
    scratch_types=[
        pltpu.VMEM((_K * _F,), jnp.float32),
        pltpu.VMEM((_NT * _CH, _C), jnp.float32),
        pltpu.VMEM((_CH * _F,), jnp.float32),
        pltpu.VMEM((160,), jnp.float32),
        pltpu.VMEM((160,), jnp.int32),
        pltpu.VMEM((96,), jnp.int32),
        pltpu.VMEM((96,), jnp.int32),
        pltpu.VMEM((96,), jnp.int32),
        pltpu.SemaphoreType.DMA,
    ],
)(_sc_body)


def kernel(tj, kernel):
    x = tj.reshape(_N, _C).astype(jnp.float32)
    xz = jnp.concatenate([x, jnp.zeros((8, _C), jnp.float32)], axis=0)
    J = kernel.reshape(_K * _F).astype(jnp.float32)
    out = _sc_kernel(xz, J)
    return out.reshape(1, _H, _W, _F).astype(jnp.float64)
